# Initial kernel scaffold; baseline (speedup 1.0000x reference)
import jax
import jax.numpy as jnp
from jax.experimental import pallas as pl


def kernel(h, edge_index, W_l, b_l, W_r, gamma, beta, running_mean, running_var):
    raise NotImplementedError("write your pallas kernel here")


# speedup vs baseline: 4.8642x; 4.8642x over previous
import functools

import jax
import jax.numpy as jnp
from jax import lax
from jax.experimental import pallas as pl
from jax.experimental.pallas import tpu as pltpu
from jax.experimental.pallas import tpu_sc as plsc

N = 10000
D = 128
E = 320000
BN_EPS = 1e-5

NC = 2
NS = 16
NW = NC * NS
EPW = E // NW
W = 80
FULL = EPW // W
NP = 10240
RPS = NP // NS
ZB = RPS // W


def _sc_aggregate(h, src, dst, zrow, onerow):
  mesh = plsc.VectorSubcoreMesh(core_axis_name="c", subcore_axis_name="s")

  @functools.partial(
      pl.kernel,
      out_type=(
          jax.ShapeDtypeStruct((NC * NP, D), jnp.float32),
          jax.ShapeDtypeStruct((NC * NP, D), jnp.float32),
      ),
      mesh=mesh,
      scratch_types=[
          pltpu.VMEM((W,), jnp.int32),
          pltpu.VMEM((W,), jnp.int32),
          pltpu.VMEM((W, D), jnp.float32),
          pltpu.VMEM((W, D), jnp.float32),
          pltpu.VMEM_SHARED((NP, D), jnp.float32),
          pltpu.SemaphoreType.DMA,
      ],
  )
  def agg_kernel(h_hbm, src_hbm, dst_hbm, zrow_hbm, onerow_hbm,
                 p_hbm, c_hbm,
                 src_v, dst_v, rows_v, ones_v, acc_sh, sem):
    cid = lax.axis_index("c")
    sid = lax.axis_index("s")
    wid = cid * NS + sid
    ebase = wid * EPW
    row0 = sid * RPS
    obase = cid * NP + row0

    pltpu.sync_copy(zrow_hbm, ones_v)
    for j in range(ZB):
      pltpu.sync_copy(ones_v, acc_sh.at[pl.ds(row0 + j * W, W)])
    plsc.subcore_barrier()

    @pl.loop(0, FULL)
    def _(i):
      b = ebase + i * W
      pltpu.sync_copy(src_hbm.at[pl.ds(b, W)], src_v)
      pltpu.sync_copy(dst_hbm.at[pl.ds(b, W)], dst_v)
      pltpu.async_copy(h_hbm.at[src_v], rows_v, sem).wait()
      pltpu.sync_copy(rows_v, acc_sh.at[dst_v], add=True)

    plsc.subcore_barrier()
    pltpu.sync_copy(acc_sh.at[pl.ds(row0, RPS)], p_hbm.at[pl.ds(obase, RPS)])

    for j in range(ZB):
      pltpu.sync_copy(ones_v, acc_sh.at[pl.ds(row0 + j * W, W)])
    pltpu.sync_copy(onerow_hbm, ones_v)
    plsc.subcore_barrier()

    @pl.loop(0, FULL)
    def _(i):
      b = ebase + i * W
      pltpu.sync_copy(dst_hbm.at[pl.ds(b, W)], dst_v)
      pltpu.sync_copy(ones_v, acc_sh.at[dst_v], add=True)

    plsc.subcore_barrier()
    pltpu.sync_copy(acc_sh.at[pl.ds(row0, RPS)], c_hbm.at[pl.ds(obase, RPS)])

  p, c = agg_kernel(h, src, dst, zrow, onerow)
  return p.reshape(NC, NP, D), c.reshape(NC, NP, D)


def _tc_body(h_ref, p_ref, c_ref, wl_ref, bl_ref, wr_ref, ga_ref, be_ref,
             mu_ref, va_ref, o_ref):
  cnt = jnp.maximum(c_ref[0, :, 0:1] + c_ref[1, :, 0:1], 1.0)
  agg = (p_ref[0] + p_ref[1]) / cnt
  hb = h_ref[...]
  dims = (((1,), (1,)), ((), ()))
  out = (lax.dot_general(agg, wl_ref[...], dims,
                         preferred_element_type=jnp.float32)
         + bl_ref[...]
         + lax.dot_general(hb, wr_ref[...], dims,
                           preferred_element_type=jnp.float32))
  s = ga_ref[...] * lax.rsqrt(va_ref[...] + BN_EPS)
  t = be_ref[...] - mu_ref[...] * s
  o_ref[...] = jnp.maximum(out * s + t, 0.0) + hb


def _tc_combine(h, p, c, W_l, b_l, W_r, gamma, beta, mu, var):
  BR = 1024
  full = lambda i: (0, 0)
  return pl.pallas_call(
      _tc_body,
      grid=(NP // BR,),
      in_specs=[
          pl.BlockSpec((BR, D), lambda i: (i, 0)),
          pl.BlockSpec((NC, BR, D), lambda i: (0, i, 0)),
          pl.BlockSpec((NC, BR, D), lambda i: (0, i, 0)),
          pl.BlockSpec((D, D), full),
          pl.BlockSpec((1, D), full),
          pl.BlockSpec((D, D), full),
          pl.BlockSpec((1, D), full),
          pl.BlockSpec((1, D), full),
          pl.BlockSpec((1, D), full),
          pl.BlockSpec((1, D), full),
      ],
      out_specs=pl.BlockSpec((BR, D), lambda i: (i, 0)),
      out_shape=jax.ShapeDtypeStruct((N, D), jnp.float32),
  )(h, p, c, W_l, b_l.reshape(1, D), W_r, gamma.reshape(1, D),
    beta.reshape(1, D), mu.reshape(1, D), var.reshape(1, D))


def kernel(h, edge_index, W_l, b_l, W_r, gamma, beta, running_mean,
           running_var):
  src = edge_index[0]
  dst = edge_index[1]
  zrow = jnp.zeros((W, D), jnp.float32)
  onerow = jnp.ones((W, D), jnp.float32)
  p, c = _sc_aggregate(h, src, dst, zrow, onerow)
  return _tc_combine(h, p, c, W_l, b_l, W_r, gamma, beta, running_mean,
                     running_var)
